# baseline (speedup 1.0000x reference)
import functools

import jax
import jax.numpy as jnp
import numpy as np
from jax import lax
from jax.experimental import pallas as pl
from jax.experimental.pallas import tpu as pltpu
from jax.experimental.pallas import tpu_sc as plsc

_N = 1000
_C = 91
_NP = 1024
_CP = 96
_RPT = 64
_DET = 100
_NF = 4096
_BS = 368
_CLIP = float(np.log(1000.0 / 16.0))
_OFF = 801.0
_BIG = np.int32(1 << 30)


def _iota16():
    return lax.broadcasted_iota(jnp.int32, (16,), 0)


def _vfull(x):
    return jnp.zeros((16,), jnp.float32) + x


def _vint(x):
    return jnp.zeros((16,), jnp.int32) + x


def _decode16(dx, dy, dw, dh, p0, p1, p2, p3):
    w = p2 - p0
    h = p3 - p1
    cx = p0 + 0.5 * w
    cy = p1 + 0.5 * h
    dx = dx / 10.0
    dy = dy / 10.0
    dw = jnp.minimum(dw / 5.0, _CLIP)
    dh = jnp.minimum(dh / 5.0, _CLIP)
    pcx = dx * w + cx
    pcy = dy * h + cy
    pw = jnp.exp(dw) * w
    ph = jnp.exp(dh) * h
    x1 = jnp.minimum(jnp.maximum(pcx - 0.5 * pw, 0.0), 800.0)
    y1 = jnp.minimum(jnp.maximum(pcy - 0.5 * ph, 0.0), 800.0)
    x2 = jnp.minimum(jnp.maximum(pcx + 0.5 * pw, 0.0), 800.0)
    y2 = jnp.minimum(jnp.maximum(pcy + 0.5 * ph, 0.0), 800.0)
    return x1, y1, x2, y2


def _body(lg_hbm, br_hbm, pr_hbm, ob_hbm, os_hbm, ol_hbm,
          lgv, brv, prv, loc, shr, stb, fil, obv, osv, olv, kidx, kflg):
    lane = _iota16()
    wid = lax.axis_index("s")
    base = wid * _RPT

    (lwk, lsc, lx1, ly1, lx2, ly2, lnx1, lny1, lnx2, lny2, llb) = loc
    (SWK, SSC, SX1, SY1, SX2, SY2, SNX1, SNY1, SNX2, SNY2, SLB) = shr
    (WK, SC, X1, Y1, X2, Y2, NX1, NY1, NX2, NY2, LB) = stb
    (FWK, FX1, FY1, FX2, FY2, FNX1, FNY1, FNX2, FNY2) = fil

    pltpu.sync_copy(lg_hbm.at[pl.ds(base * _CP, _RPT * _CP)], lgv)
    pltpu.sync_copy(br_hbm.at[pl.ds(base * _BS, _RPT * _BS)], brv)
    pltpu.sync_copy(pr_hbm.at[pl.ds(base * 4, _RPT * 4)], prv)

    for g in range(_RPT // 16):
        li = g * 16 + lane
        rowoff = li * _CP

        def p1(jc, m):
            for u in range(7):
                c = 7 * jc + u
                v = plsc.load_gather(lgv, [rowoff + c])
                m = jnp.maximum(m, v)
            return m
        m = lax.fori_loop(0, _C // 7, p1, jnp.zeros((16,), jnp.float32) - 3e38)

        def p2(jc, carry):
            tot, best, bc = carry
            for u in range(7):
                c = 7 * jc + u
                v = plsc.load_gather(lgv, [rowoff + c])
                e = jnp.exp(v - m)
                better = jnp.logical_and(e > best, c >= 1)
                tot = tot + e
                best = jnp.where(better, e, best)
                bc = jnp.where(better, c, bc)
            return tot, best, bc
        tot, best, bc = lax.fori_loop(
            0, _C // 7, p2,
            (jnp.zeros((16,), jnp.float32),
             jnp.zeros((16,), jnp.float32) - 1.0,
             jnp.zeros((16,), jnp.int32)))

        score = best / tot
        valid = score > 0.5
        sl = pl.ds(g * 16, 16)
        lwk[sl] = jnp.where(valid, score, -3.0)
        lsc[sl] = score
        llb[sl] = bc

    for g in range(_RPT // 16):
        li = g * 16 + lane
        sl = pl.ds(g * 16, 16)
        ri = li * _BS + 4 * llb[sl]
        dx = plsc.load_gather(brv, [ri])
        dy = plsc.load_gather(brv, [ri + 1])
        dw = plsc.load_gather(brv, [ri + 2])
        dh = plsc.load_gather(brv, [ri + 3])
        p0 = plsc.load_gather(prv, [li * 4])
        p1v = plsc.load_gather(prv, [li * 4 + 1])
        p2v = plsc.load_gather(prv, [li * 4 + 2])
        p3v = plsc.load_gather(prv, [li * 4 + 3])
        x1, y1, x2, y2 = _decode16(dx, dy, dw, dh, p0, p1v, p2v, p3v)
        off = llb[sl].astype(jnp.float32) * _OFF
        lx1[sl] = x1
        ly1[sl] = y1
        lx2[sl] = x2
        ly2[sl] = y2
        lnx1[sl] = x1 + off
        lny1[sl] = y1 + off
        lnx2[sl] = x2 + off
        lny2[sl] = y2 + off

    dst = pl.ds(base, _RPT)
    pltpu.sync_copy(lwk, SWK.at[dst])
    pltpu.sync_copy(lsc, SSC.at[dst])
    pltpu.sync_copy(lx1, SX1.at[dst])
    pltpu.sync_copy(ly1, SY1.at[dst])
    pltpu.sync_copy(lx2, SX2.at[dst])
    pltpu.sync_copy(ly2, SY2.at[dst])
    pltpu.sync_copy(lnx1, SNX1.at[dst])
    pltpu.sync_copy(lny1, SNY1.at[dst])
    pltpu.sync_copy(lnx2, SNX2.at[dst])
    pltpu.sync_copy(lny2, SNY2.at[dst])
    pltpu.sync_copy(llb, SLB.at[dst])

    plsc.subcore_barrier()

    @pl.when(wid == 0)
    def _stage2():
        pltpu.sync_copy(SWK, WK)
        pltpu.sync_copy(SSC, SC)
        pltpu.sync_copy(SX1, X1)
        pltpu.sync_copy(SY1, Y1)
        pltpu.sync_copy(SX2, X2)
        pltpu.sync_copy(SY2, Y2)
        pltpu.sync_copy(SNX1, NX1)
        pltpu.sync_copy(SNY1, NY1)
        pltpu.sync_copy(SNX2, NX2)
        pltpu.sync_copy(SNY2, NY2)
        pltpu.sync_copy(SLB, LB)

        def iou_over_half(b0, b1, b2, b3, a1, qx1, qy1, qx2, qy2):
            a2 = (qx2 - qx1) * (qy2 - qy1)
            ltx = jnp.maximum(b0, qx1)
            lty = jnp.maximum(b1, qy1)
            rbx = jnp.minimum(b2, qx2)
            rby = jnp.minimum(b3, qy2)
            wx = jnp.maximum(rbx - ltx, 0.0)
            wy = jnp.maximum(rby - lty, 0.0)
            inter = wx * wy
            return (inter + inter) > (a1 + a2 - inter + 1e-9)

        def record(k, idxv, flagv):
            plsc.store_scatter(kidx, [_vint(k)], idxv, mask=lane == 0)
            plsc.store_scatter(kflg, [_vint(k)], flagv, mask=lane == 0)

        for q in range(8):
            cs = pl.ds(q * 16, 16)
            kidx[cs] = jnp.zeros((16,), jnp.int32)
            kflg[cs] = jnp.ones((16,), jnp.int32)

        def fused_pass(b0, b1, b2, b3):
            a1 = (b2 - b0) * (b3 - b1)

            def fp(j, carry):
                vm, vi = carry
                for u in range(4):
                    cj = 4 * j + u
                    cs = pl.ds(cj * 16, 16)
                    sp = iou_over_half(b0, b1, b2, b3, a1,
                                       NX1[cs], NY1[cs], NX2[cs], NY2[cs])
                    w = jnp.where(sp, -3.0, WK[cs])
                    WK[cs] = w
                    b = w > vm
                    vm = jnp.where(b, w, vm)
                    vi = jnp.where(b, cj * 16 + lane, vi)
                return vm, vi
            vm, vi = lax.fori_loop(
                0, _NP // 64, fp,
                (jnp.zeros((16,), jnp.float32) - 9.0, jnp.zeros((16,), jnp.int32)),
                unroll=2)
            mx = jnp.max(vm)
            return mx, jnp.min(jnp.where(vm == mx, vi, _BIG))

        dummy = _vfull(-1e9)
        mx0, i0 = fused_pass(dummy, dummy, dummy, dummy)

        def iter_body(k, carry):
            mx, i, ready = carry
            valid = mx > 0.0
            ii = _vint(i)

            @pl.when(valid)
            def _pick():
                record(k, ii, _vint(1))

            @pl.when(jnp.logical_not(valid))
            def _filler():
                @pl.when(ready == 0)
                def _init():
                    def fb(q, _):
                        f = q * 16 + lane
                        n = f // (_C - 1)
                        c = f % (_C - 1) + 1
                        rsc = plsc.load_gather(SC, [n])
                        rlb = plsc.load_gather(LB, [n])
                        isval = jnp.logical_and(rlb == c, rsc > 0.5)
                        FWK[pl.ds(q * 16, 16)] = jnp.where(isval, -3.0, -1.0)
                        return 0
                    lax.fori_loop(0, _NF // 16, fb, 0)

                    def fd(q, _):
                        f = q * 16 + lane
                        n = f // (_C - 1)
                        c = f % (_C - 1) + 1
                        ri = n * _BS + 4 * c
                        dx = plsc.load_gather(brv, [ri])
                        dy = plsc.load_gather(brv, [ri + 1])
                        dw = plsc.load_gather(brv, [ri + 2])
                        dh = plsc.load_gather(brv, [ri + 3])
                        p0 = plsc.load_gather(prv, [n * 4])
                        p1v = plsc.load_gather(prv, [n * 4 + 1])
                        p2v = plsc.load_gather(prv, [n * 4 + 2])
                        p3v = plsc.load_gather(prv, [n * 4 + 3])
                        x1, y1, x2, y2 = _decode16(dx, dy, dw, dh, p0, p1v, p2v, p3v)
                        off = c.astype(jnp.float32) * _OFF
                        cs = pl.ds(q * 16, 16)
                        FX1[cs] = x1
                        FY1[cs] = y1
                        FX2[cs] = x2
                        FY2[cs] = y2
                        FNX1[cs] = x1 + off
                        FNY1[cs] = y1 + off
                        FNX2[cs] = x2 + off
                        FNY2[cs] = y2 + off
                        return 0
                    lax.fori_loop(0, _NF // 16, fd, 0)

                    def ps(j, _):
                        kj = plsc.load_gather(kidx, [_vint(j)])
                        b0 = plsc.load_gather(NX1, [kj])
                        b1 = plsc.load_gather(NY1, [kj])
                        b2 = plsc.load_gather(NX2, [kj])
                        b3 = plsc.load_gather(NY2, [kj])
                        a1 = (b2 - b0) * (b3 - b1)

                        def s2(q, _):
                            cs = pl.ds(q * 16, 16)
                            sp = iou_over_half(b0, b1, b2, b3, a1,
                                               FNX1[cs], FNY1[cs], FNX2[cs], FNY2[cs])
                            FWK[cs] = jnp.where(sp, -3.0, FWK[cs])
                            return 0
                        lax.fori_loop(0, _NF // 16, s2, 0)
                        return 0
                    lax.fori_loop(0, k, ps, 0)

                def fm(q, fmin):
                    v = FWK[pl.ds(q * 16, 16)]
                    cnd = jnp.where(v > -2.0, q * 16 + lane, _BIG)
                    return jnp.minimum(fmin, jnp.min(cnd))
                fstar = lax.fori_loop(0, _NF // 16, fm, _BIG)

                @pl.when(fstar < _BIG)
                def _fpick():
                    ff = _vint(fstar)
                    record(k, ff, _vint(0))
                    b0 = plsc.load_gather(FNX1, [ff])
                    b1 = plsc.load_gather(FNY1, [ff])
                    b2 = plsc.load_gather(FNX2, [ff])
                    b3 = plsc.load_gather(FNY2, [ff])
                    a1 = (b2 - b0) * (b3 - b1)

                    def s3(q, _):
                        cs = pl.ds(q * 16, 16)
                        sp = iou_over_half(b0, b1, b2, b3, a1,
                                           FNX1[cs], FNY1[cs], FNX2[cs], FNY2[cs])
                        sp = jnp.logical_or(sp, (q * 16 + lane) == fstar)
                        FWK[cs] = jnp.where(sp, -3.0, FWK[cs])
                        return 0
                    lax.fori_loop(0, _NF // 16, s3, 0)

                @pl.when(fstar >= _BIG)
                def _fexhausted():
                    record(k, _vint(0), _vint(2))

            new_ready = jnp.where(valid, ready, jnp.int32(1))
            b0 = jnp.where(valid, plsc.load_gather(NX1, [ii]), dummy)
            b1 = jnp.where(valid, plsc.load_gather(NY1, [ii]), dummy)
            b2 = jnp.where(valid, plsc.load_gather(NX2, [ii]), dummy)
            b3 = jnp.where(valid, plsc.load_gather(NY2, [ii]), dummy)
            plsc.store_scatter(WK, [ii], _vfull(-3.0), mask=lane == 0)
            mx2, i2 = fused_pass(b0, b1, b2, b3)
            return mx2, i2, new_ready

        lax.fori_loop(0, _DET, iter_body, (mx0, i0, jnp.int32(0)))

        for q in range(7):
            cs = pl.ds(q * 16, 16)
            kpos = q * 16 + lane
            ki = kidx[cs]
            fl = kflg[cs]
            kc = jnp.minimum(ki, _NP - 1)
            isc = fl == 1
            x1o = jnp.where(isc, plsc.load_gather(X1, [kc]), plsc.load_gather(FX1, [ki]))
            y1o = jnp.where(isc, plsc.load_gather(Y1, [kc]), plsc.load_gather(FY1, [ki]))
            x2o = jnp.where(isc, plsc.load_gather(X2, [kc]), plsc.load_gather(FX2, [ki]))
            y2o = jnp.where(isc, plsc.load_gather(Y2, [kc]), plsc.load_gather(FY2, [ki]))
            sco = jnp.where(isc, plsc.load_gather(SC, [kc]), 0.0)
            lbo = jnp.where(isc, plsc.load_gather(LB, [kc]), ki % (_C - 1) + 1)
            plsc.store_scatter(obv, [4 * kpos], x1o)
            plsc.store_scatter(obv, [4 * kpos + 1], y1o)
            plsc.store_scatter(obv, [4 * kpos + 2], x2o)
            plsc.store_scatter(obv, [4 * kpos + 3], y2o)
            osv[cs] = sco
            olv[cs] = lbo

        z = _vint(0)
        ob00 = plsc.load_gather(obv, [z])
        ob01 = plsc.load_gather(obv, [z + 1])
        ob02 = plsc.load_gather(obv, [z + 2])
        ob03 = plsc.load_gather(obv, [z + 3])
        os0 = plsc.load_gather(osv, [z])
        ol0 = plsc.load_gather(olv, [z])
        for q in range(7):
            cs = pl.ds(q * 16, 16)
            kpos = q * 16 + lane
            m2 = kflg[cs] == 2
            plsc.store_scatter(obv, [4 * kpos], ob00, mask=m2)
            plsc.store_scatter(obv, [4 * kpos + 1], ob01, mask=m2)
            plsc.store_scatter(obv, [4 * kpos + 2], ob02, mask=m2)
            plsc.store_scatter(obv, [4 * kpos + 3], ob03, mask=m2)
            plsc.store_scatter(osv, [kpos], os0, mask=m2)
            plsc.store_scatter(olv, [kpos], ol0, mask=m2)

        pltpu.sync_copy(obv, ob_hbm)
        pltpu.sync_copy(osv, os_hbm)
        pltpu.sync_copy(olv, ol_hbm)


_f32 = jnp.float32
_i32 = jnp.int32

_nms_call = pl.kernel(
    _body,
    out_type=(
        jax.ShapeDtypeStruct((512,), _f32),
        jax.ShapeDtypeStruct((128,), _f32),
        jax.ShapeDtypeStruct((128,), _i32),
    ),
    mesh=plsc.VectorSubcoreMesh(core_axis_name="c", subcore_axis_name="s",
                                num_cores=1),
    compiler_params=pltpu.CompilerParams(needs_layout_passes=False),
    scratch_types=(
        pltpu.VMEM((_RPT * _CP,), _f32),
        pltpu.VMEM((_RPT * _BS,), _f32),
        pltpu.VMEM((_RPT * 4,), _f32),
        tuple([pltpu.VMEM((_RPT,), _f32)] * 10 + [pltpu.VMEM((_RPT,), _i32)]),
        tuple([pltpu.VMEM_SHARED((_NP,), _f32)] * 10
              + [pltpu.VMEM_SHARED((_NP,), _i32)]),
        tuple([pltpu.VMEM((_NP,), _f32)] * 10 + [pltpu.VMEM((_NP,), _i32)]),
        tuple([pltpu.VMEM((_NF,), _f32)] * 9),
        pltpu.VMEM((512,), _f32),
        pltpu.VMEM((128,), _f32),
        pltpu.VMEM((128,), _i32),
        pltpu.VMEM((128,), _i32),
        pltpu.VMEM((128,), _i32),
    ),
)


@jax.jit
def kernel(class_logits, box_regression, proposals):
    lg = jnp.full((_NP, _CP), -1e30, _f32).at[:_N, :_C].set(class_logits)
    br = jnp.zeros((_NP, _BS), _f32).at[:_N, :_C * 4].set(box_regression)
    pr = jnp.zeros((_NP * 4,), _f32).at[:_N * 4].set(proposals.reshape(-1))
    ob, osc, olb = _nms_call(lg.reshape(-1), br.reshape(-1), pr)
    return ob[:4 * _DET].reshape(_DET, 4), osc[:_DET], olb[:_DET]


# speedup vs baseline: 1.7908x; 1.7908x over previous
import functools

import jax
import jax.numpy as jnp
import numpy as np
from jax import lax
from jax.experimental import pallas as pl
from jax.experimental.pallas import tpu as pltpu
from jax.experimental.pallas import tpu_sc as plsc

_N = 1000
_C = 91
_NP = 1024
_CP = 96
_RPT = 64
_DET = 100
_NF = 4096
_BS = 368
_CLIP = float(np.log(1000.0 / 16.0))
_OFF = 801.0
_BIG = np.int32(1 << 30)


def _iota16():
    return lax.broadcasted_iota(jnp.int32, (16,), 0)


def _vfull(x):
    return jnp.zeros((16,), jnp.float32) + x


def _vint(x):
    return jnp.zeros((16,), jnp.int32) + x


def _decode16(dx, dy, dw, dh, p0, p1, p2, p3):
    w = p2 - p0
    h = p3 - p1
    cx = p0 + 0.5 * w
    cy = p1 + 0.5 * h
    dx = dx / 10.0
    dy = dy / 10.0
    dw = jnp.minimum(dw / 5.0, _CLIP)
    dh = jnp.minimum(dh / 5.0, _CLIP)
    pcx = dx * w + cx
    pcy = dy * h + cy
    pw = jnp.exp(dw) * w
    ph = jnp.exp(dh) * h
    x1 = jnp.minimum(jnp.maximum(pcx - 0.5 * pw, 0.0), 800.0)
    y1 = jnp.minimum(jnp.maximum(pcy - 0.5 * ph, 0.0), 800.0)
    x2 = jnp.minimum(jnp.maximum(pcx + 0.5 * pw, 0.0), 800.0)
    y2 = jnp.minimum(jnp.maximum(pcy + 0.5 * ph, 0.0), 800.0)
    return x1, y1, x2, y2


def _body(lg_hbm, br_hbm, pr_hbm, ob_hbm, os_hbm, ol_hbm,
          lgv, brv, prv, loc, shr, stb, fil, obv, osv, olv, kidx, kflg, sem):
    lane = _iota16()
    wid = lax.axis_index("s")
    base = wid * _RPT

    (lwk, lsc, lx1, ly1, lx2, ly2, lnx1, lny1, lnx2, lny2, llb) = loc
    (SWK, SSC, SX1, SY1, SX2, SY2, SNX1, SNY1, SNX2, SNY2, SLB) = shr
    (WK, SC, X1, Y1, X2, Y2, NX1, NY1, NX2, NY2, LB) = stb
    (FWK, FX1, FY1, FX2, FY2, FNX1, FNY1, FNX2, FNY2) = fil

    br_cp = pltpu.async_copy(br_hbm.at[pl.ds(base * _BS, _RPT * _BS)], brv, sem)
    pr_cp = pltpu.async_copy(pr_hbm.at[pl.ds(base * 4, _RPT * 4)], prv, sem)
    pltpu.sync_copy(lg_hbm.at[pl.ds(base * _CP, _RPT * _CP)], lgv)

    for g in range(_RPT // 16):
        li = g * 16 + lane
        rowoff = li * _CP

        def p1(jc, m):
            for u in range(7):
                c = 7 * jc + u
                v = plsc.load_gather(lgv, [rowoff + c])
                m = jnp.maximum(m, v)
            return m
        m = lax.fori_loop(0, _C // 7, p1, jnp.zeros((16,), jnp.float32) - 3e38)

        def p2(jc, carry):
            tot, best, bc = carry
            for u in range(7):
                c = 7 * jc + u
                v = plsc.load_gather(lgv, [rowoff + c])
                e = jnp.exp(v - m)
                better = jnp.logical_and(e > best, c >= 1)
                tot = tot + e
                best = jnp.where(better, e, best)
                bc = jnp.where(better, c, bc)
            return tot, best, bc
        tot, best, bc = lax.fori_loop(
            0, _C // 7, p2,
            (jnp.zeros((16,), jnp.float32),
             jnp.zeros((16,), jnp.float32) - 1.0,
             jnp.zeros((16,), jnp.int32)))

        score = best / tot
        valid = score > 0.5
        sl = pl.ds(g * 16, 16)
        lwk[sl] = jnp.where(valid, score, -3.0)
        lsc[sl] = score
        llb[sl] = bc

    br_cp.wait()
    pr_cp.wait()

    for g in range(_RPT // 16):
        li = g * 16 + lane
        sl = pl.ds(g * 16, 16)
        ri = li * _BS + 4 * llb[sl]
        dx = plsc.load_gather(brv, [ri])
        dy = plsc.load_gather(brv, [ri + 1])
        dw = plsc.load_gather(brv, [ri + 2])
        dh = plsc.load_gather(brv, [ri + 3])
        p0 = plsc.load_gather(prv, [li * 4])
        p1v = plsc.load_gather(prv, [li * 4 + 1])
        p2v = plsc.load_gather(prv, [li * 4 + 2])
        p3v = plsc.load_gather(prv, [li * 4 + 3])
        x1, y1, x2, y2 = _decode16(dx, dy, dw, dh, p0, p1v, p2v, p3v)
        off = llb[sl].astype(jnp.float32) * _OFF
        lx1[sl] = x1
        ly1[sl] = y1
        lx2[sl] = x2
        ly2[sl] = y2
        lnx1[sl] = x1 + off
        lny1[sl] = y1 + off
        lnx2[sl] = x2 + off
        lny2[sl] = y2 + off

    dst = pl.ds(base, _RPT)
    pltpu.sync_copy(lwk, SWK.at[dst])
    pltpu.sync_copy(lsc, SSC.at[dst])
    pltpu.sync_copy(lx1, SX1.at[dst])
    pltpu.sync_copy(ly1, SY1.at[dst])
    pltpu.sync_copy(lx2, SX2.at[dst])
    pltpu.sync_copy(ly2, SY2.at[dst])
    pltpu.sync_copy(lnx1, SNX1.at[dst])
    pltpu.sync_copy(lny1, SNY1.at[dst])
    pltpu.sync_copy(lnx2, SNX2.at[dst])
    pltpu.sync_copy(lny2, SNY2.at[dst])
    pltpu.sync_copy(llb, SLB.at[dst])

    plsc.subcore_barrier()

    @pl.when(wid == 0)
    def _stage2():
        cps = [pltpu.async_copy(a, b, sem) for a, b in
               ((SWK, WK), (SSC, SC), (SX1, X1), (SY1, Y1), (SX2, X2),
                (SY2, Y2), (SNX1, NX1), (SNY1, NY1), (SNX2, NX2),
                (SNY2, NY2), (SLB, LB))]
        for c in cps:
            c.wait()

        def iou_over_half(b0, b1, b2, b3, a1, qx1, qy1, qx2, qy2):
            a2 = (qx2 - qx1) * (qy2 - qy1)
            ltx = jnp.maximum(b0, qx1)
            lty = jnp.maximum(b1, qy1)
            rbx = jnp.minimum(b2, qx2)
            rby = jnp.minimum(b3, qy2)
            wx = jnp.maximum(rbx - ltx, 0.0)
            wy = jnp.maximum(rby - lty, 0.0)
            inter = wx * wy
            return (inter + inter) > (a1 + a2 - inter + 1e-9)

        def record(k, idxv, flagv):
            plsc.store_scatter(kidx, [_vint(k)], idxv, mask=lane == 0)
            plsc.store_scatter(kflg, [_vint(k)], flagv, mask=lane == 0)

        for q in range(8):
            cs = pl.ds(q * 16, 16)
            kidx[cs] = jnp.zeros((16,), jnp.int32)
            kflg[cs] = jnp.ones((16,), jnp.int32)

        def fused_pass(b0, b1, b2, b3):
            a1 = (b2 - b0) * (b3 - b1)

            def fp(j, carry):
                vm, vi = carry
                for u in range(4):
                    cj = 4 * j + u
                    cs = pl.ds(cj * 16, 16)
                    sp = iou_over_half(b0, b1, b2, b3, a1,
                                       NX1[cs], NY1[cs], NX2[cs], NY2[cs])
                    w = jnp.where(sp, -3.0, WK[cs])
                    WK[cs] = w
                    b = w > vm
                    vm = jnp.where(b, w, vm)
                    vi = jnp.where(b, cj * 16 + lane, vi)
                return vm, vi
            vm, vi = lax.fori_loop(
                0, _NP // 64, fp,
                (jnp.zeros((16,), jnp.float32) - 9.0, jnp.zeros((16,), jnp.int32)))
            mx = jnp.max(vm)
            return mx, jnp.min(jnp.where(vm == mx, vi, _BIG))

        dummy = _vfull(-1e9)
        mx0, i0 = fused_pass(dummy, dummy, dummy, dummy)

        def iter_body(k, carry):
            mx, i, ready = carry
            valid = mx > 0.0
            ii = _vint(i)

            @pl.when(valid)
            def _pick():
                record(k, ii, _vint(1))

            @pl.when(jnp.logical_not(valid))
            def _filler():
                @pl.when(ready == 0)
                def _init():
                    def fb(q, _):
                        f = q * 16 + lane
                        n = f // (_C - 1)
                        c = f % (_C - 1) + 1
                        rsc = plsc.load_gather(SC, [n])
                        rlb = plsc.load_gather(LB, [n])
                        isval = jnp.logical_and(rlb == c, rsc > 0.5)
                        FWK[pl.ds(q * 16, 16)] = jnp.where(isval, -3.0, -1.0)
                        return 0
                    lax.fori_loop(0, _NF // 16, fb, 0)

                    def fd(q, _):
                        f = q * 16 + lane
                        n = f // (_C - 1)
                        c = f % (_C - 1) + 1
                        ri = n * _BS + 4 * c
                        dx = plsc.load_gather(brv, [ri])
                        dy = plsc.load_gather(brv, [ri + 1])
                        dw = plsc.load_gather(brv, [ri + 2])
                        dh = plsc.load_gather(brv, [ri + 3])
                        p0 = plsc.load_gather(prv, [n * 4])
                        p1v = plsc.load_gather(prv, [n * 4 + 1])
                        p2v = plsc.load_gather(prv, [n * 4 + 2])
                        p3v = plsc.load_gather(prv, [n * 4 + 3])
                        x1, y1, x2, y2 = _decode16(dx, dy, dw, dh, p0, p1v, p2v, p3v)
                        off = c.astype(jnp.float32) * _OFF
                        cs = pl.ds(q * 16, 16)
                        FX1[cs] = x1
                        FY1[cs] = y1
                        FX2[cs] = x2
                        FY2[cs] = y2
                        FNX1[cs] = x1 + off
                        FNY1[cs] = y1 + off
                        FNX2[cs] = x2 + off
                        FNY2[cs] = y2 + off
                        return 0
                    lax.fori_loop(0, _NF // 16, fd, 0)

                    def ps(j, _):
                        kj = plsc.load_gather(kidx, [_vint(j)])
                        b0 = plsc.load_gather(NX1, [kj])
                        b1 = plsc.load_gather(NY1, [kj])
                        b2 = plsc.load_gather(NX2, [kj])
                        b3 = plsc.load_gather(NY2, [kj])
                        a1 = (b2 - b0) * (b3 - b1)

                        def s2(q, _):
                            cs = pl.ds(q * 16, 16)
                            sp = iou_over_half(b0, b1, b2, b3, a1,
                                               FNX1[cs], FNY1[cs], FNX2[cs], FNY2[cs])
                            FWK[cs] = jnp.where(sp, -3.0, FWK[cs])
                            return 0
                        lax.fori_loop(0, _NF // 16, s2, 0)
                        return 0
                    lax.fori_loop(0, k, ps, 0)

                def fm(q, fmin):
                    v = FWK[pl.ds(q * 16, 16)]
                    cnd = jnp.where(v > -2.0, q * 16 + lane, _BIG)
                    return jnp.minimum(fmin, jnp.min(cnd))
                fstar = lax.fori_loop(0, _NF // 16, fm, _BIG)

                @pl.when(fstar < _BIG)
                def _fpick():
                    ff = _vint(fstar)
                    record(k, ff, _vint(0))
                    b0 = plsc.load_gather(FNX1, [ff])
                    b1 = plsc.load_gather(FNY1, [ff])
                    b2 = plsc.load_gather(FNX2, [ff])
                    b3 = plsc.load_gather(FNY2, [ff])
                    a1 = (b2 - b0) * (b3 - b1)

                    def s3(q, _):
                        cs = pl.ds(q * 16, 16)
                        sp = iou_over_half(b0, b1, b2, b3, a1,
                                           FNX1[cs], FNY1[cs], FNX2[cs], FNY2[cs])
                        sp = jnp.logical_or(sp, (q * 16 + lane) == fstar)
                        FWK[cs] = jnp.where(sp, -3.0, FWK[cs])
                        return 0
                    lax.fori_loop(0, _NF // 16, s3, 0)

                @pl.when(fstar >= _BIG)
                def _fexhausted():
                    record(k, _vint(0), _vint(2))

            new_ready = jnp.where(valid, ready, jnp.int32(1))
            b0 = jnp.where(valid, plsc.load_gather(NX1, [ii]), dummy)
            b1 = jnp.where(valid, plsc.load_gather(NY1, [ii]), dummy)
            b2 = jnp.where(valid, plsc.load_gather(NX2, [ii]), dummy)
            b3 = jnp.where(valid, plsc.load_gather(NY2, [ii]), dummy)
            plsc.store_scatter(WK, [ii], _vfull(-3.0), mask=lane == 0)
            mx2, i2 = fused_pass(b0, b1, b2, b3)
            return mx2, i2, new_ready

        lax.fori_loop(0, _DET, iter_body, (mx0, i0, jnp.int32(0)))

        for q in range(7):
            cs = pl.ds(q * 16, 16)
            kpos = q * 16 + lane
            ki = kidx[cs]
            fl = kflg[cs]
            kc = jnp.minimum(ki, _NP - 1)
            isc = fl == 1
            x1o = jnp.where(isc, plsc.load_gather(X1, [kc]), plsc.load_gather(FX1, [ki]))
            y1o = jnp.where(isc, plsc.load_gather(Y1, [kc]), plsc.load_gather(FY1, [ki]))
            x2o = jnp.where(isc, plsc.load_gather(X2, [kc]), plsc.load_gather(FX2, [ki]))
            y2o = jnp.where(isc, plsc.load_gather(Y2, [kc]), plsc.load_gather(FY2, [ki]))
            sco = jnp.where(isc, plsc.load_gather(SC, [kc]), 0.0)
            lbo = jnp.where(isc, plsc.load_gather(LB, [kc]), ki % (_C - 1) + 1)
            plsc.store_scatter(obv, [4 * kpos], x1o)
            plsc.store_scatter(obv, [4 * kpos + 1], y1o)
            plsc.store_scatter(obv, [4 * kpos + 2], x2o)
            plsc.store_scatter(obv, [4 * kpos + 3], y2o)
            osv[cs] = sco
            olv[cs] = lbo

        z = _vint(0)
        ob00 = plsc.load_gather(obv, [z])
        ob01 = plsc.load_gather(obv, [z + 1])
        ob02 = plsc.load_gather(obv, [z + 2])
        ob03 = plsc.load_gather(obv, [z + 3])
        os0 = plsc.load_gather(osv, [z])
        ol0 = plsc.load_gather(olv, [z])
        for q in range(7):
            cs = pl.ds(q * 16, 16)
            kpos = q * 16 + lane
            m2 = kflg[cs] == 2
            plsc.store_scatter(obv, [4 * kpos], ob00, mask=m2)
            plsc.store_scatter(obv, [4 * kpos + 1], ob01, mask=m2)
            plsc.store_scatter(obv, [4 * kpos + 2], ob02, mask=m2)
            plsc.store_scatter(obv, [4 * kpos + 3], ob03, mask=m2)
            plsc.store_scatter(osv, [kpos], os0, mask=m2)
            plsc.store_scatter(olv, [kpos], ol0, mask=m2)

        pltpu.sync_copy(obv, ob_hbm)
        pltpu.sync_copy(osv, os_hbm)
        pltpu.sync_copy(olv, ol_hbm)


_f32 = jnp.float32
_i32 = jnp.int32

_nms_call = pl.kernel(
    _body,
    out_type=(
        jax.ShapeDtypeStruct((512,), _f32),
        jax.ShapeDtypeStruct((128,), _f32),
        jax.ShapeDtypeStruct((128,), _i32),
    ),
    mesh=plsc.VectorSubcoreMesh(core_axis_name="c", subcore_axis_name="s",
                                num_cores=1),
    compiler_params=pltpu.CompilerParams(needs_layout_passes=False),
    scratch_types=(
        pltpu.VMEM((_RPT * _CP,), _f32),
        pltpu.VMEM((_RPT * _BS,), _f32),
        pltpu.VMEM((_RPT * 4,), _f32),
        tuple([pltpu.VMEM((_RPT,), _f32)] * 10 + [pltpu.VMEM((_RPT,), _i32)]),
        tuple([pltpu.VMEM_SHARED((_NP,), _f32)] * 10
              + [pltpu.VMEM_SHARED((_NP,), _i32)]),
        tuple([pltpu.VMEM((_NP,), _f32)] * 10 + [pltpu.VMEM((_NP,), _i32)]),
        tuple([pltpu.VMEM((_NF,), _f32)] * 9),
        pltpu.VMEM((512,), _f32),
        pltpu.VMEM((128,), _f32),
        pltpu.VMEM((128,), _i32),
        pltpu.VMEM((128,), _i32),
        pltpu.VMEM((128,), _i32),
        pltpu.SemaphoreType.DMA,
    ),
)


@jax.jit
def kernel(class_logits, box_regression, proposals):
    lg = jnp.full((_NP, _CP), -1e30, _f32).at[:_N, :_C].set(class_logits)
    br = jnp.zeros((_NP, _BS), _f32).at[:_N, :_C * 4].set(box_regression)
    pr = jnp.zeros((_NP * 4,), _f32).at[:_N * 4].set(proposals.reshape(-1))
    ob, osc, olb = _nms_call(lg.reshape(-1), br.reshape(-1), pr)
    return ob[:4 * _DET].reshape(_DET, 4), osc[:_DET], olb[:_DET]


# speedup vs baseline: 1.8112x; 1.0114x over previous
import functools

import jax
import jax.numpy as jnp
import numpy as np
from jax import lax
from jax.experimental import pallas as pl
from jax.experimental.pallas import tpu as pltpu
from jax.experimental.pallas import tpu_sc as plsc

_N = 1000
_C = 91
_NP = 1024
_CP = 96
_RPT = 64
_DET = 100
_NF = 4096
_BS = 368
_CLIP = float(np.log(1000.0 / 16.0))
_OFF = 801.0
_BIG = np.int32(1 << 30)


def _iota16():
    return lax.broadcasted_iota(jnp.int32, (16,), 0)


def _vfull(x):
    return jnp.zeros((16,), jnp.float32) + x


def _vint(x):
    return jnp.zeros((16,), jnp.int32) + x


def _decode16(dx, dy, dw, dh, p0, p1, p2, p3):
    w = p2 - p0
    h = p3 - p1
    cx = p0 + 0.5 * w
    cy = p1 + 0.5 * h
    dx = dx / 10.0
    dy = dy / 10.0
    dw = jnp.minimum(dw / 5.0, _CLIP)
    dh = jnp.minimum(dh / 5.0, _CLIP)
    pcx = dx * w + cx
    pcy = dy * h + cy
    pw = jnp.exp(dw) * w
    ph = jnp.exp(dh) * h
    x1 = jnp.minimum(jnp.maximum(pcx - 0.5 * pw, 0.0), 800.0)
    y1 = jnp.minimum(jnp.maximum(pcy - 0.5 * ph, 0.0), 800.0)
    x2 = jnp.minimum(jnp.maximum(pcx + 0.5 * pw, 0.0), 800.0)
    y2 = jnp.minimum(jnp.maximum(pcy + 0.5 * ph, 0.0), 800.0)
    return x1, y1, x2, y2


def _body(lg_hbm, br_hbm, pr_hbm, ob_hbm, os_hbm, ol_hbm,
          lgv, brv, prv, loc, shr, stb, fil, obv, osv, olv, kidx, kflg, sem):
    lane = _iota16()
    wid = lax.axis_index("s")
    base = wid * _RPT

    (lwk, lsc, lx1, ly1, lx2, ly2, lnx1, lny1, lnx2, lny2, llb) = loc
    (SWK, SSC, SX1, SY1, SX2, SY2, SNX1, SNY1, SNX2, SNY2, SLB) = shr
    (WK, SC, X1, Y1, X2, Y2, NX1, NY1, NX2, NY2, LB) = stb
    (FWK, FX1, FY1, FX2, FY2, FNX1, FNY1, FNX2, FNY2) = fil

    br_cp = pltpu.async_copy(br_hbm.at[pl.ds(base * _BS, _RPT * _BS)], brv, sem)
    pr_cp = pltpu.async_copy(pr_hbm.at[pl.ds(base * 4, _RPT * 4)], prv, sem)
    pltpu.sync_copy(lg_hbm.at[pl.ds(base * _CP, _RPT * _CP)], lgv)

    for g in range(_RPT // 16):
        li = g * 16 + lane
        rowoff = li * _CP

        def p1(jc, m):
            for u in range(7):
                c = 7 * jc + u
                v = plsc.load_gather(lgv, [rowoff + c])
                m = jnp.maximum(m, v)
            return m
        m = lax.fori_loop(0, _C // 7, p1, jnp.zeros((16,), jnp.float32) - 3e38)

        def p2(jc, carry):
            tot, best, bc = carry
            for u in range(7):
                c = 7 * jc + u
                v = plsc.load_gather(lgv, [rowoff + c])
                e = jnp.exp(v - m)
                better = jnp.logical_and(e > best, c >= 1)
                tot = tot + e
                best = jnp.where(better, e, best)
                bc = jnp.where(better, c, bc)
            return tot, best, bc
        tot, best, bc = lax.fori_loop(
            0, _C // 7, p2,
            (jnp.zeros((16,), jnp.float32),
             jnp.zeros((16,), jnp.float32) - 1.0,
             jnp.zeros((16,), jnp.int32)))

        score = best / tot
        valid = score > 0.5
        sl = pl.ds(g * 16, 16)
        lwk[sl] = jnp.where(valid, score, -3.0)
        lsc[sl] = score
        llb[sl] = bc

    br_cp.wait()
    pr_cp.wait()

    for g in range(_RPT // 16):
        li = g * 16 + lane
        sl = pl.ds(g * 16, 16)
        ri = li * _BS + 4 * llb[sl]
        dx = plsc.load_gather(brv, [ri])
        dy = plsc.load_gather(brv, [ri + 1])
        dw = plsc.load_gather(brv, [ri + 2])
        dh = plsc.load_gather(brv, [ri + 3])
        p0 = plsc.load_gather(prv, [li * 4])
        p1v = plsc.load_gather(prv, [li * 4 + 1])
        p2v = plsc.load_gather(prv, [li * 4 + 2])
        p3v = plsc.load_gather(prv, [li * 4 + 3])
        x1, y1, x2, y2 = _decode16(dx, dy, dw, dh, p0, p1v, p2v, p3v)
        off = llb[sl].astype(jnp.float32) * _OFF
        lx1[sl] = x1
        ly1[sl] = y1
        lx2[sl] = x2
        ly2[sl] = y2
        lnx1[sl] = x1 + off
        lny1[sl] = y1 + off
        lnx2[sl] = x2 + off
        lny2[sl] = y2 + off

    dst = pl.ds(base, _RPT)
    pubs = [pltpu.async_copy(a, b.at[dst], sem) for a, b in
            ((lwk, SWK), (lsc, SSC), (lx1, SX1), (ly1, SY1), (lx2, SX2),
             (ly2, SY2), (lnx1, SNX1), (lny1, SNY1), (lnx2, SNX2),
             (lny2, SNY2), (llb, SLB))]
    for c in pubs:
        c.wait()

    plsc.subcore_barrier()

    @pl.when(wid == 0)
    def _stage2():
        cps = [pltpu.async_copy(a, b, sem) for a, b in
               ((SWK, WK), (SSC, SC), (SX1, X1), (SY1, Y1), (SX2, X2),
                (SY2, Y2), (SNX1, NX1), (SNY1, NY1), (SNX2, NX2),
                (SNY2, NY2), (SLB, LB))]
        for c in cps:
            c.wait()

        def iou_over_half(b0, b1, b2, b3, a1, qx1, qy1, qx2, qy2):
            a2 = (qx2 - qx1) * (qy2 - qy1)
            ltx = jnp.maximum(b0, qx1)
            lty = jnp.maximum(b1, qy1)
            rbx = jnp.minimum(b2, qx2)
            rby = jnp.minimum(b3, qy2)
            wx = jnp.maximum(rbx - ltx, 0.0)
            wy = jnp.maximum(rby - lty, 0.0)
            inter = wx * wy
            return (inter + inter) > (a1 + a2 - inter + 1e-9)

        def record(k, idxv, flagv):
            plsc.store_scatter(kidx, [_vint(k)], idxv, mask=lane == 0)
            plsc.store_scatter(kflg, [_vint(k)], flagv, mask=lane == 0)

        for q in range(8):
            cs = pl.ds(q * 16, 16)
            kidx[cs] = jnp.zeros((16,), jnp.int32)
            kflg[cs] = jnp.ones((16,), jnp.int32)

        def fused_pass(b0, b1, b2, b3):
            a1 = (b2 - b0) * (b3 - b1)

            def fp(j, carry):
                vm, vi = carry
                for u in range(4):
                    cj = 4 * j + u
                    cs = pl.ds(cj * 16, 16)
                    sp = iou_over_half(b0, b1, b2, b3, a1,
                                       NX1[cs], NY1[cs], NX2[cs], NY2[cs])
                    w = jnp.where(sp, -3.0, WK[cs])
                    WK[cs] = w
                    b = w > vm
                    vm = jnp.where(b, w, vm)
                    vi = jnp.where(b, cj * 16 + lane, vi)
                return vm, vi
            vm, vi = lax.fori_loop(
                0, _NP // 64, fp,
                (jnp.zeros((16,), jnp.float32) - 9.0, jnp.zeros((16,), jnp.int32)))
            mx = jnp.max(vm)
            return mx, jnp.min(jnp.where(vm == mx, vi, _BIG))

        dummy = _vfull(-1e9)
        mx0, i0 = fused_pass(dummy, dummy, dummy, dummy)

        def iter_body(k, carry):
            mx, i, ready = carry
            valid = mx > 0.0
            ii = _vint(i)

            @pl.when(valid)
            def _pick():
                record(k, ii, _vint(1))

            @pl.when(jnp.logical_not(valid))
            def _filler():
                @pl.when(ready == 0)
                def _init():
                    def fb(q, _):
                        f = q * 16 + lane
                        n = f // (_C - 1)
                        c = f % (_C - 1) + 1
                        rsc = plsc.load_gather(SC, [n])
                        rlb = plsc.load_gather(LB, [n])
                        isval = jnp.logical_and(rlb == c, rsc > 0.5)
                        FWK[pl.ds(q * 16, 16)] = jnp.where(isval, -3.0, -1.0)
                        return 0
                    lax.fori_loop(0, _NF // 16, fb, 0)

                    def fd(q, _):
                        f = q * 16 + lane
                        n = f // (_C - 1)
                        c = f % (_C - 1) + 1
                        ri = n * _BS + 4 * c
                        dx = plsc.load_gather(brv, [ri])
                        dy = plsc.load_gather(brv, [ri + 1])
                        dw = plsc.load_gather(brv, [ri + 2])
                        dh = plsc.load_gather(brv, [ri + 3])
                        p0 = plsc.load_gather(prv, [n * 4])
                        p1v = plsc.load_gather(prv, [n * 4 + 1])
                        p2v = plsc.load_gather(prv, [n * 4 + 2])
                        p3v = plsc.load_gather(prv, [n * 4 + 3])
                        x1, y1, x2, y2 = _decode16(dx, dy, dw, dh, p0, p1v, p2v, p3v)
                        off = c.astype(jnp.float32) * _OFF
                        cs = pl.ds(q * 16, 16)
                        FX1[cs] = x1
                        FY1[cs] = y1
                        FX2[cs] = x2
                        FY2[cs] = y2
                        FNX1[cs] = x1 + off
                        FNY1[cs] = y1 + off
                        FNX2[cs] = x2 + off
                        FNY2[cs] = y2 + off
                        return 0
                    lax.fori_loop(0, _NF // 16, fd, 0)

                    def ps(j, _):
                        kj = plsc.load_gather(kidx, [_vint(j)])
                        b0 = plsc.load_gather(NX1, [kj])
                        b1 = plsc.load_gather(NY1, [kj])
                        b2 = plsc.load_gather(NX2, [kj])
                        b3 = plsc.load_gather(NY2, [kj])
                        a1 = (b2 - b0) * (b3 - b1)

                        def s2(q, _):
                            cs = pl.ds(q * 16, 16)
                            sp = iou_over_half(b0, b1, b2, b3, a1,
                                               FNX1[cs], FNY1[cs], FNX2[cs], FNY2[cs])
                            FWK[cs] = jnp.where(sp, -3.0, FWK[cs])
                            return 0
                        lax.fori_loop(0, _NF // 16, s2, 0)
                        return 0
                    lax.fori_loop(0, k, ps, 0)

                def fm(q, fmin):
                    v = FWK[pl.ds(q * 16, 16)]
                    cnd = jnp.where(v > -2.0, q * 16 + lane, _BIG)
                    return jnp.minimum(fmin, jnp.min(cnd))
                fstar = lax.fori_loop(0, _NF // 16, fm, _BIG)

                @pl.when(fstar < _BIG)
                def _fpick():
                    ff = _vint(fstar)
                    record(k, ff, _vint(0))
                    b0 = plsc.load_gather(FNX1, [ff])
                    b1 = plsc.load_gather(FNY1, [ff])
                    b2 = plsc.load_gather(FNX2, [ff])
                    b3 = plsc.load_gather(FNY2, [ff])
                    a1 = (b2 - b0) * (b3 - b1)

                    def s3(q, _):
                        cs = pl.ds(q * 16, 16)
                        sp = iou_over_half(b0, b1, b2, b3, a1,
                                           FNX1[cs], FNY1[cs], FNX2[cs], FNY2[cs])
                        sp = jnp.logical_or(sp, (q * 16 + lane) == fstar)
                        FWK[cs] = jnp.where(sp, -3.0, FWK[cs])
                        return 0
                    lax.fori_loop(0, _NF // 16, s3, 0)

                @pl.when(fstar >= _BIG)
                def _fexhausted():
                    record(k, _vint(0), _vint(2))

            new_ready = jnp.where(valid, ready, jnp.int32(1))
            b0 = jnp.where(valid, plsc.load_gather(NX1, [ii]), dummy)
            b1 = jnp.where(valid, plsc.load_gather(NY1, [ii]), dummy)
            b2 = jnp.where(valid, plsc.load_gather(NX2, [ii]), dummy)
            b3 = jnp.where(valid, plsc.load_gather(NY2, [ii]), dummy)
            plsc.store_scatter(WK, [ii], _vfull(-3.0), mask=lane == 0)
            mx2, i2 = fused_pass(b0, b1, b2, b3)
            return mx2, i2, new_ready

        lax.fori_loop(0, _DET, iter_body, (mx0, i0, jnp.int32(0)))

        for q in range(7):
            cs = pl.ds(q * 16, 16)
            kpos = q * 16 + lane
            ki = kidx[cs]
            fl = kflg[cs]
            kc = jnp.minimum(ki, _NP - 1)
            isc = fl == 1
            x1o = jnp.where(isc, plsc.load_gather(X1, [kc]), plsc.load_gather(FX1, [ki]))
            y1o = jnp.where(isc, plsc.load_gather(Y1, [kc]), plsc.load_gather(FY1, [ki]))
            x2o = jnp.where(isc, plsc.load_gather(X2, [kc]), plsc.load_gather(FX2, [ki]))
            y2o = jnp.where(isc, plsc.load_gather(Y2, [kc]), plsc.load_gather(FY2, [ki]))
            sco = jnp.where(isc, plsc.load_gather(SC, [kc]), 0.0)
            lbo = jnp.where(isc, plsc.load_gather(LB, [kc]), ki % (_C - 1) + 1)
            plsc.store_scatter(obv, [4 * kpos], x1o)
            plsc.store_scatter(obv, [4 * kpos + 1], y1o)
            plsc.store_scatter(obv, [4 * kpos + 2], x2o)
            plsc.store_scatter(obv, [4 * kpos + 3], y2o)
            osv[cs] = sco
            olv[cs] = lbo

        z = _vint(0)
        ob00 = plsc.load_gather(obv, [z])
        ob01 = plsc.load_gather(obv, [z + 1])
        ob02 = plsc.load_gather(obv, [z + 2])
        ob03 = plsc.load_gather(obv, [z + 3])
        os0 = plsc.load_gather(osv, [z])
        ol0 = plsc.load_gather(olv, [z])
        for q in range(7):
            cs = pl.ds(q * 16, 16)
            kpos = q * 16 + lane
            m2 = kflg[cs] == 2
            plsc.store_scatter(obv, [4 * kpos], ob00, mask=m2)
            plsc.store_scatter(obv, [4 * kpos + 1], ob01, mask=m2)
            plsc.store_scatter(obv, [4 * kpos + 2], ob02, mask=m2)
            plsc.store_scatter(obv, [4 * kpos + 3], ob03, mask=m2)
            plsc.store_scatter(osv, [kpos], os0, mask=m2)
            plsc.store_scatter(olv, [kpos], ol0, mask=m2)

        outs = [pltpu.async_copy(a, b, sem) for a, b in
                ((obv, ob_hbm), (osv, os_hbm), (olv, ol_hbm))]
        for c in outs:
            c.wait()


_f32 = jnp.float32
_i32 = jnp.int32

_nms_call = pl.kernel(
    _body,
    out_type=(
        jax.ShapeDtypeStruct((512,), _f32),
        jax.ShapeDtypeStruct((128,), _f32),
        jax.ShapeDtypeStruct((128,), _i32),
    ),
    mesh=plsc.VectorSubcoreMesh(core_axis_name="c", subcore_axis_name="s",
                                num_cores=1),
    compiler_params=pltpu.CompilerParams(needs_layout_passes=False),
    scratch_types=(
        pltpu.VMEM((_RPT * _CP,), _f32),
        pltpu.VMEM((_RPT * _BS,), _f32),
        pltpu.VMEM((_RPT * 4,), _f32),
        tuple([pltpu.VMEM((_RPT,), _f32)] * 10 + [pltpu.VMEM((_RPT,), _i32)]),
        tuple([pltpu.VMEM_SHARED((_NP,), _f32)] * 10
              + [pltpu.VMEM_SHARED((_NP,), _i32)]),
        tuple([pltpu.VMEM((_NP,), _f32)] * 10 + [pltpu.VMEM((_NP,), _i32)]),
        tuple([pltpu.VMEM((_NF,), _f32)] * 9),
        pltpu.VMEM((512,), _f32),
        pltpu.VMEM((128,), _f32),
        pltpu.VMEM((128,), _i32),
        pltpu.VMEM((128,), _i32),
        pltpu.VMEM((128,), _i32),
        pltpu.SemaphoreType.DMA,
    ),
)


@jax.jit
def kernel(class_logits, box_regression, proposals):
    lg = jnp.full((_NP, _CP), -1e30, _f32).at[:_N, :_C].set(class_logits)
    br = jnp.zeros((_NP, _BS), _f32).at[:_N, :_C * 4].set(box_regression)
    pr = jnp.zeros((_NP * 4,), _f32).at[:_N * 4].set(proposals.reshape(-1))
    ob, osc, olb = _nms_call(lg.reshape(-1), br.reshape(-1), pr)
    return ob[:4 * _DET].reshape(_DET, 4), osc[:_DET], olb[:_DET]
